# baseline (speedup 1.0000x reference)
import jax
import jax.numpy as jnp
from jax import lax
from jax.experimental import pallas as pl
from jax.experimental.pallas import tpu as pltpu
from jax.experimental.pallas import tpu_sc as plsc

B = 16384
D = 64
NC = 2
NS = 16
NW = NC * NS
BPW = B // NW
CHUNK = 128
NCHUNK = BPW // CHUNK
GROUPS = BPW // 16


def _diff_body(uid_hbm, pid_hbm, nid_hbm, utab_hbm, itab_hbm, ibias_hbm,
               diff_hbm, uidx, pidx, nidx, urows, prows, nrows, pb, nb,
               diffv, sem):
    wid = lax.axis_index("s") * NC + lax.axis_index("c")
    base = wid * BPW
    pltpu.sync_copy(uid_hbm.at[pl.ds(base, BPW)], uidx)
    pltpu.sync_copy(pid_hbm.at[pl.ds(base, BPW)], pidx)
    pltpu.sync_copy(nid_hbm.at[pl.ds(base, BPW)], nidx)
    copies = []
    for j in range(NCHUNK):
        sl = pl.ds(j * CHUNK, CHUNK)
        copies.append(pltpu.async_copy(utab_hbm.at[uidx.at[sl]], urows.at[sl], sem))
        copies.append(pltpu.async_copy(itab_hbm.at[pidx.at[sl]], prows.at[sl], sem))
        copies.append(pltpu.async_copy(itab_hbm.at[nidx.at[sl]], nrows.at[sl], sem))
        copies.append(pltpu.async_copy(ibias_hbm.at[pidx.at[sl]], pb.at[sl], sem))
        copies.append(pltpu.async_copy(ibias_hbm.at[nidx.at[sl]], nb.at[sl], sem))
    for c in copies:
        c.wait()

    def group_body(g, carry):
        r16 = g * 16 + lax.iota(jnp.int32, 16)
        zero16 = jnp.zeros((16,), jnp.int32)
        acc = jnp.zeros((16,), jnp.float32)
        for d in range(D):
            d16 = jnp.full((16,), d, jnp.int32)
            uu = plsc.load_gather(urows, [r16, d16])
            pp = plsc.load_gather(prows, [r16, d16])
            nn = plsc.load_gather(nrows, [r16, d16])
            acc = acc + uu * (pp - nn)
        pb16 = plsc.load_gather(pb, [r16, zero16])
        nb16 = plsc.load_gather(nb, [r16, zero16])
        plsc.store_scatter(diffv, [r16], acc + pb16 - nb16)
        return carry

    lax.fori_loop(0, GROUPS, group_body, 0)
    pltpu.sync_copy(diffv, diff_hbm.at[pl.ds(base, BPW)])


_diff_call = pl.kernel(
    _diff_body,
    out_type=jax.ShapeDtypeStruct((B,), jnp.float32),
    mesh=plsc.VectorSubcoreMesh(core_axis_name="c", subcore_axis_name="s",
                                num_cores=NC, num_subcores=NS),
    scratch_types=[
        pltpu.VMEM((BPW,), jnp.int32),
        pltpu.VMEM((BPW,), jnp.int32),
        pltpu.VMEM((BPW,), jnp.int32),
        pltpu.VMEM((BPW, D), jnp.float32),
        pltpu.VMEM((BPW, D), jnp.float32),
        pltpu.VMEM((BPW, D), jnp.float32),
        pltpu.VMEM((BPW, 1), jnp.float32),
        pltpu.VMEM((BPW, 1), jnp.float32),
        pltpu.VMEM((BPW,), jnp.float32),
        pltpu.SemaphoreType.DMA,
    ],
    compiler_params=pltpu.CompilerParams(needs_layout_passes=False,
                                         use_tc_tiling_on_sc=False),
)


def _loss_body(diff_ref, out_ref):
    x = diff_ref[...]
    sp = jnp.maximum(-x, 0.0) + jnp.log1p(jnp.exp(-jnp.abs(x)))
    out_ref[...] = jnp.sum(sp).reshape(1, 1)


_loss_call = pl.pallas_call(
    _loss_body,
    out_shape=jax.ShapeDtypeStruct((1, 1), jnp.float32),
)


def kernel(user_id, p_item_id, n_item_id, user_table, item_table, item_bias):
    uid = user_id.astype(jnp.int32)
    pid = p_item_id.astype(jnp.int32)
    nid = n_item_id.astype(jnp.int32)
    diff = _diff_call(uid, pid, nid, user_table, item_table, item_bias)
    loss = _loss_call(diff.reshape(B // 128, 128))
    return loss[0, 0]


# speedup vs baseline: 1.0041x; 1.0041x over previous
import jax
import jax.numpy as jnp
from jax import lax
from jax.experimental import pallas as pl
from jax.experimental.pallas import tpu as pltpu
from jax.experimental.pallas import tpu_sc as plsc

B = 16384
D = 64
NC = 2
NS = 16
NW = NC * NS
BPW = B // NW
CHUNK = 128
NCHUNK = BPW // CHUNK
GPC = CHUNK // 16


def _dot_body(uid_hbm, pid_hbm, nid_hbm, utab_hbm, itab_hbm, diff_hbm,
              uidx, pidx, nidx, uphys, pphys, nphys,
              ubuf, pbuf, nbuf, diffv, sem):
    wid = lax.axis_index("s") * NC + lax.axis_index("c")
    base = wid * BPW
    pltpu.sync_copy(uid_hbm.at[pl.ds(base, BPW)], uidx)
    pltpu.sync_copy(pid_hbm.at[pl.ds(base, BPW)], pidx)
    pltpu.sync_copy(nid_hbm.at[pl.ds(base, BPW)], nidx)

    def shift_body(i, carry):
        sl = pl.ds(i * 16, 16)
        uphys[sl] = uidx[sl] >> 1
        pphys[sl] = pidx[sl] >> 1
        nphys[sl] = nidx[sl] >> 1
        return carry

    lax.fori_loop(0, BPW // 16, shift_body, 0)

    def chunk_gather(j):
        sl = pl.ds(j * CHUNK, CHUNK)
        return [
            pltpu.async_copy(utab_hbm.at[uphys.at[sl]], ubuf, sem),
            pltpu.async_copy(itab_hbm.at[pphys.at[sl]], pbuf, sem),
            pltpu.async_copy(itab_hbm.at[nphys.at[sl]], nbuf, sem),
        ]

    for j in range(NCHUNK):
        copies = chunk_gather(j)
        for c in copies:
            c.wait()

        def group_body(g, carry):
            loc16 = g * 16 + lax.iota(jnp.int32, 16)
            glob = pl.ds(j * CHUNK + g * 16, 16)
            ucol = (uidx[glob] & 1) * D
            pcol = (pidx[glob] & 1) * D
            ncol = (nidx[glob] & 1) * D
            acc = jnp.zeros((16,), jnp.float32)
            for d in range(D):
                uu = plsc.load_gather(ubuf, [loc16, ucol + d])
                pp = plsc.load_gather(pbuf, [loc16, pcol + d])
                nn = plsc.load_gather(nbuf, [loc16, ncol + d])
                acc = acc + uu * (pp - nn)
            diffv[pl.ds(j * CHUNK + g * 16, 16)] = acc
            return carry

        lax.fori_loop(0, GPC, group_body, 0)

    pltpu.sync_copy(diffv, diff_hbm.at[pl.ds(base, BPW)])


_dot_call = pl.kernel(
    _dot_body,
    out_type=jax.ShapeDtypeStruct((B,), jnp.float32),
    mesh=plsc.VectorSubcoreMesh(core_axis_name="c", subcore_axis_name="s",
                                num_cores=NC, num_subcores=NS),
    scratch_types=[
        pltpu.VMEM((BPW,), jnp.int32),
        pltpu.VMEM((BPW,), jnp.int32),
        pltpu.VMEM((BPW,), jnp.int32),
        pltpu.VMEM((BPW,), jnp.int32),
        pltpu.VMEM((BPW,), jnp.int32),
        pltpu.VMEM((BPW,), jnp.int32),
        pltpu.VMEM((CHUNK, 2 * D), jnp.float32),
        pltpu.VMEM((CHUNK, 2 * D), jnp.float32),
        pltpu.VMEM((CHUNK, 2 * D), jnp.float32),
        pltpu.VMEM((BPW,), jnp.float32),
        pltpu.SemaphoreType.DMA,
    ],
    compiler_params=pltpu.CompilerParams(needs_layout_passes=False),
)


def _bias_body(pid_hbm, nid_hbm, ibias_hbm, bdiff_hbm,
               pidx, nidx, pb, nb, bdiffv, sem):
    wid = lax.axis_index("s") * NC + lax.axis_index("c")
    base = wid * BPW
    pltpu.sync_copy(pid_hbm.at[pl.ds(base, BPW)], pidx)
    pltpu.sync_copy(nid_hbm.at[pl.ds(base, BPW)], nidx)
    copies = []
    for j in range(NCHUNK):
        sl = pl.ds(j * CHUNK, CHUNK)
        copies.append(pltpu.async_copy(ibias_hbm.at[pidx.at[sl]], pb.at[sl], sem))
        copies.append(pltpu.async_copy(ibias_hbm.at[nidx.at[sl]], nb.at[sl], sem))
    for c in copies:
        c.wait()

    def group_body(g, carry):
        r16 = g * 16 + lax.iota(jnp.int32, 16)
        zero16 = jnp.zeros((16,), jnp.int32)
        pb16 = plsc.load_gather(pb, [r16, zero16])
        nb16 = plsc.load_gather(nb, [r16, zero16])
        plsc.store_scatter(bdiffv, [r16], pb16 - nb16)
        return carry

    lax.fori_loop(0, BPW // 16, group_body, 0)
    pltpu.sync_copy(bdiffv, bdiff_hbm.at[pl.ds(base, BPW)])


_bias_call = pl.kernel(
    _bias_body,
    out_type=jax.ShapeDtypeStruct((B,), jnp.float32),
    mesh=plsc.VectorSubcoreMesh(core_axis_name="c", subcore_axis_name="s",
                                num_cores=NC, num_subcores=NS),
    scratch_types=[
        pltpu.VMEM((BPW,), jnp.int32),
        pltpu.VMEM((BPW,), jnp.int32),
        pltpu.VMEM((BPW, 1), jnp.float32),
        pltpu.VMEM((BPW, 1), jnp.float32),
        pltpu.VMEM((BPW,), jnp.float32),
        pltpu.SemaphoreType.DMA,
    ],
    compiler_params=pltpu.CompilerParams(needs_layout_passes=False,
                                         use_tc_tiling_on_sc=False),
)


def _loss_body(dd_ref, bd_ref, out_ref):
    x = dd_ref[...] + bd_ref[...]
    sp = jnp.maximum(-x, 0.0) + jnp.log1p(jnp.exp(-jnp.abs(x)))
    out_ref[...] = jnp.sum(sp).reshape(1, 1)


_loss_call = pl.pallas_call(
    _loss_body,
    out_shape=jax.ShapeDtypeStruct((1, 1), jnp.float32),
)


def kernel(user_id, p_item_id, n_item_id, user_table, item_table, item_bias):
    uid = user_id.astype(jnp.int32)
    pid = p_item_id.astype(jnp.int32)
    nid = n_item_id.astype(jnp.int32)
    utab2 = user_table.reshape(user_table.shape[0] // 2, 2 * D)
    itab2 = item_table.reshape(item_table.shape[0] // 2, 2 * D)
    diff_dot = _dot_call(uid, pid, nid, utab2, itab2)
    bdiff = _bias_call(pid, nid, item_bias)
    loss = _loss_call(diff_dot.reshape(B // 128, 128),
                      bdiff.reshape(B // 128, 128))
    return loss[0, 0]


# speedup vs baseline: 1.7501x; 1.7429x over previous
import jax
import jax.numpy as jnp
from jax import lax
from jax.experimental import pallas as pl
from jax.experimental.pallas import tpu as pltpu
from jax.experimental.pallas import tpu_sc as plsc

B = 16384
D = 64
NC = 2
NS = 16
NW = NC * NS
BPW = B // NW
CH = 16
NCH = BPW // CH


CHUNK = 128
NCHUNK = BPW // CHUNK
GPC = CHUNK // 16


def _dot_body(uid_hbm, pid_hbm, nid_hbm, utab_hbm, itab_hbm, diff_hbm,
              uidx, pidx, nidx, uphys, pphys, nphys,
              ubuf, pbuf, nbuf, diffv, sem):
    wid = lax.axis_index("s") * NC + lax.axis_index("c")
    base = wid * BPW
    pltpu.sync_copy(uid_hbm.at[pl.ds(base, BPW)], uidx)
    pltpu.sync_copy(pid_hbm.at[pl.ds(base, BPW)], pidx)
    pltpu.sync_copy(nid_hbm.at[pl.ds(base, BPW)], nidx)

    def shift_body(i, carry):
        sl = pl.ds(i * 16, 16)
        uphys[sl] = uidx[sl] >> 1
        pphys[sl] = pidx[sl] >> 1
        nphys[sl] = nidx[sl] >> 1
        return carry

    lax.fori_loop(0, BPW // 16, shift_body, 0)

    iota16 = lax.iota(jnp.int32, 16)
    for j in range(NCHUNK):
        sl = pl.ds(j * CHUNK, CHUNK)
        copies = [
            pltpu.async_copy(utab_hbm.at[uphys.at[sl]], ubuf, sem),
            pltpu.async_copy(itab_hbm.at[pphys.at[sl]], pbuf, sem),
            pltpu.async_copy(itab_hbm.at[nphys.at[sl]], nbuf, sem),
        ]
        for c in copies:
            c.wait()

        def group_body(g, carry):
            loc16 = g * 16 + iota16
            glob = pl.ds(j * CHUNK + g * 16, 16)
            ucol = (uidx[glob] & 1) * D
            pcol = (pidx[glob] & 1) * D
            ncol = (nidx[glob] & 1) * D
            acc = jnp.zeros((16,), jnp.float32)
            for d in range(D):
                uu = plsc.load_gather(ubuf, [loc16, ucol + d])
                pp = plsc.load_gather(pbuf, [loc16, pcol + d])
                nn = plsc.load_gather(nbuf, [loc16, ncol + d])
                acc = acc + uu * (pp - nn)
            diffv[pl.ds(j * CHUNK + g * 16, 16)] = acc
            return carry

        lax.fori_loop(0, GPC, group_body, 0)

    pltpu.sync_copy(diffv, diff_hbm.at[pl.ds(base, BPW)])


_diff_call = pl.kernel(
    _dot_body,
    out_type=jax.ShapeDtypeStruct((B,), jnp.float32),
    mesh=plsc.VectorSubcoreMesh(core_axis_name="c", subcore_axis_name="s",
                                num_cores=NC, num_subcores=NS),
    scratch_types=[
        pltpu.VMEM((BPW,), jnp.int32),
        pltpu.VMEM((BPW,), jnp.int32),
        pltpu.VMEM((BPW,), jnp.int32),
        pltpu.VMEM((BPW,), jnp.int32),
        pltpu.VMEM((BPW,), jnp.int32),
        pltpu.VMEM((BPW,), jnp.int32),
        pltpu.VMEM((CHUNK, 2 * D), jnp.float32),
        pltpu.VMEM((CHUNK, 2 * D), jnp.float32),
        pltpu.VMEM((CHUNK, 2 * D), jnp.float32),
        pltpu.VMEM((BPW,), jnp.float32),
        pltpu.SemaphoreType.DMA,
    ],
    compiler_params=pltpu.CompilerParams(needs_layout_passes=False),
)


def _bias_body(pid_hbm, nid_hbm, ibias_hbm, bdiff_hbm,
               pidx, nidx, pb, nb, bdiffv, sem):
    wid = lax.axis_index("s") * NC + lax.axis_index("c")
    base = wid * BPW
    pltpu.sync_copy(pid_hbm.at[pl.ds(base, BPW)], pidx)
    pltpu.sync_copy(nid_hbm.at[pl.ds(base, BPW)], nidx)
    copies = []
    for j in range(BPW // 128):
        sl = pl.ds(j * 128, 128)
        copies.append(pltpu.async_copy(ibias_hbm.at[pidx.at[sl]], pb.at[sl], sem))
        copies.append(pltpu.async_copy(ibias_hbm.at[nidx.at[sl]], nb.at[sl], sem))
    for c in copies:
        c.wait()

    def group_body(g, carry):
        sl = pl.ds(g * 16, 16)
        bdiffv[sl] = pb[sl] - nb[sl]
        return carry

    lax.fori_loop(0, BPW // 16, group_body, 0)
    pltpu.sync_copy(bdiffv, bdiff_hbm.at[pl.ds(base, BPW)])


_bias_call = pl.kernel(
    _bias_body,
    out_type=jax.ShapeDtypeStruct((B,), jnp.float32),
    mesh=plsc.VectorSubcoreMesh(core_axis_name="c", subcore_axis_name="s",
                                num_cores=NC, num_subcores=NS),
    scratch_types=[
        pltpu.VMEM((BPW,), jnp.int32),
        pltpu.VMEM((BPW,), jnp.int32),
        pltpu.VMEM((BPW,), jnp.float32),
        pltpu.VMEM((BPW,), jnp.float32),
        pltpu.VMEM((BPW,), jnp.float32),
        pltpu.SemaphoreType.DMA,
    ],
    compiler_params=pltpu.CompilerParams(needs_layout_passes=False,
                                         use_tc_tiling_on_sc=False),
)


def _loss_body(diff_ref, bd_ref, out_ref):
    x = diff_ref[...] + bd_ref[...]
    sp = jnp.maximum(-x, 0.0) + jnp.log1p(jnp.exp(-jnp.abs(x)))
    out_ref[...] = jnp.sum(sp).reshape(1, 1)


_loss_call = pl.pallas_call(
    _loss_body,
    out_shape=jax.ShapeDtypeStruct((1, 1), jnp.float32),
)


def kernel(user_id, p_item_id, n_item_id, user_table, item_table, item_bias):
    uid = user_id.astype(jnp.int32)
    pid = p_item_id.astype(jnp.int32)
    nid = n_item_id.astype(jnp.int32)
    utab2 = user_table.reshape(user_table.shape[0] // 2, 2 * D)
    itab2 = item_table.reshape(item_table.shape[0] // 2, 2 * D)
    diff = _diff_call(uid, pid, nid, utab2, itab2)
    bdiff = _bias_call(pid, nid, item_bias.reshape(-1))
    loss = _loss_call(diff.reshape(B // 128, 128),
                      bdiff.reshape(B // 128, 128))
    return loss[0, 0]


# speedup vs baseline: 1.8135x; 1.0362x over previous
import jax
import jax.numpy as jnp
from jax import lax
from jax.experimental import pallas as pl
from jax.experimental.pallas import tpu as pltpu
from jax.experimental.pallas import tpu_sc as plsc

B = 16384
D = 64
NC = 2
NS = 16
NW = NC * NS
BPW = B // NW
CHUNK = 128
NCHUNK = BPW // CHUNK

_MESH = plsc.VectorSubcoreMesh(core_axis_name="c", subcore_axis_name="s",
                               num_cores=NC, num_subcores=NS)
_SC_PARAMS = pltpu.CompilerParams(needs_layout_passes=False,
                                  use_tc_tiling_on_sc=False)


def _worker_base():
    return (lax.axis_index("s") * NC + lax.axis_index("c")) * BPW


def _u_body(uid_hbm, utab_hbm, urows_hbm, uidx, ubuf, sem):
    base = _worker_base()
    pltpu.sync_copy(uid_hbm.at[pl.ds(base, BPW)], uidx)
    copies = [
        pltpu.async_copy(utab_hbm.at[uidx.at[pl.ds(j * CHUNK, CHUNK)]],
                         ubuf.at[pl.ds(j * CHUNK, CHUNK)], sem)
        for j in range(NCHUNK)
    ]
    for c in copies:
        c.wait()
    pltpu.sync_copy(ubuf, urows_hbm.at[pl.ds(base, BPW)])


_u_call = pl.kernel(
    _u_body,
    out_type=jax.ShapeDtypeStruct((B, D), jnp.float32),
    mesh=_MESH,
    scratch_types=[
        pltpu.VMEM((BPW,), jnp.int32),
        pltpu.VMEM((BPW, D), jnp.float32),
        pltpu.SemaphoreType.DMA,
    ],
    compiler_params=_SC_PARAMS,
)


def _pn_body(pid_hbm, nid_hbm, itab_hbm, ibias_hbm, pn_hbm, bdiff_hbm,
             pidx, nidx, pbuf, nbuf, pb, nb, bdiffv, sem):
    base = _worker_base()
    pltpu.sync_copy(pid_hbm.at[pl.ds(base, BPW)], pidx)
    pltpu.sync_copy(nid_hbm.at[pl.ds(base, BPW)], nidx)
    copies = []
    for j in range(NCHUNK):
        sl = pl.ds(j * CHUNK, CHUNK)
        copies.append(pltpu.async_copy(itab_hbm.at[pidx.at[sl]],
                                       pbuf.at[sl], sem))
        copies.append(pltpu.async_copy(itab_hbm.at[nidx.at[sl]],
                                       nbuf.at[sl], sem))
        copies.append(pltpu.async_copy(ibias_hbm.at[pidx.at[sl]],
                                       pb.at[sl], sem))
        copies.append(pltpu.async_copy(ibias_hbm.at[nidx.at[sl]],
                                       nb.at[sl], sem))
    for c in copies:
        c.wait()

    def sub_body(r, carry):
        for c in range(D // 16):
            sl = pl.ds(c * 16, 16)
            pbuf[r, sl] = pbuf[r, sl] - nbuf[r, sl]
        return carry

    lax.fori_loop(0, BPW, sub_body, 0)

    def bias_body(i, carry):
        sl = pl.ds(i * 16, 16)
        bdiffv[sl] = pb[sl] - nb[sl]
        return carry

    lax.fori_loop(0, BPW // 16, bias_body, 0)
    pltpu.sync_copy(pbuf, pn_hbm.at[pl.ds(base, BPW)])
    pltpu.sync_copy(bdiffv, bdiff_hbm.at[pl.ds(base, BPW)])


_pn_call = pl.kernel(
    _pn_body,
    out_type=(jax.ShapeDtypeStruct((B, D), jnp.float32),
              jax.ShapeDtypeStruct((B,), jnp.float32)),
    mesh=_MESH,
    scratch_types=[
        pltpu.VMEM((BPW,), jnp.int32),
        pltpu.VMEM((BPW,), jnp.int32),
        pltpu.VMEM((BPW, D), jnp.float32),
        pltpu.VMEM((BPW, D), jnp.float32),
        pltpu.VMEM((BPW,), jnp.float32),
        pltpu.VMEM((BPW,), jnp.float32),
        pltpu.VMEM((BPW,), jnp.float32),
        pltpu.SemaphoreType.DMA,
    ],
    compiler_params=_SC_PARAMS,
)


def _loss_body(u_ref, pn_ref, bd_ref, out_ref):
    prod = u_ref[...] * pn_ref[...]
    bd = bd_ref[...]
    s0 = jnp.sum(prod[:, :D], axis=1) + bd[:, 0]
    s1 = jnp.sum(prod[:, D:], axis=1) + bd[:, 1]
    sp0 = jnp.maximum(-s0, 0.0) + jnp.log1p(jnp.exp(-jnp.abs(s0)))
    sp1 = jnp.maximum(-s1, 0.0) + jnp.log1p(jnp.exp(-jnp.abs(s1)))
    out_ref[...] = (jnp.sum(sp0) + jnp.sum(sp1)).reshape(1, 1)


_loss_call = pl.pallas_call(
    _loss_body,
    out_shape=jax.ShapeDtypeStruct((1, 1), jnp.float32),
)


def kernel(user_id, p_item_id, n_item_id, user_table, item_table, item_bias):
    uid = user_id.astype(jnp.int32)
    pid = p_item_id.astype(jnp.int32)
    nid = n_item_id.astype(jnp.int32)
    urows = _u_call(uid, user_table)
    pn, bdiff = _pn_call(pid, nid, item_table, item_bias.reshape(-1))
    loss = _loss_call(urows.reshape(B // 2, 2 * D),
                      pn.reshape(B // 2, 2 * D),
                      bdiff.reshape(B // 2, 2))
    return loss[0, 0]


# speedup vs baseline: 2.9113x; 1.6054x over previous
import jax
import jax.numpy as jnp
from jax import lax
from jax.experimental import pallas as pl
from jax.experimental.pallas import tpu as pltpu
from jax.experimental.pallas import tpu_sc as plsc

B = 16384
D = 64
NC = 2
NS = 16
NW = NC * NS
BPW = B // NW
WAVE = 4

_MESH = plsc.VectorSubcoreMesh(core_axis_name="c", subcore_axis_name="s",
                               num_cores=NC, num_subcores=NS)


def _worker_base():
    return (lax.axis_index("s") * NC + lax.axis_index("c")) * BPW


def _dot_body(uid_hbm, pid_hbm, nid_hbm, utabT_hbm, itabT_hbm, diff_hbm,
              uidx, pidx, nidx, ubuf, pbuf, nbuf, diffv, sem):
    base = _worker_base()
    pltpu.sync_copy(uid_hbm.at[pl.ds(base, BPW)], uidx)
    pltpu.sync_copy(pid_hbm.at[pl.ds(base, BPW)], pidx)
    pltpu.sync_copy(nid_hbm.at[pl.ds(base, BPW)], nidx)
    iota16 = lax.iota(jnp.int32, 16)

    def group_body(g, carry):
        uv = uidx[pl.ds(g * 16, 16)]
        pv = pidx[pl.ds(g * 16, 16)]
        nv = nidx[pl.ds(g * 16, 16)]
        diff16 = jnp.zeros((16,), jnp.float32)
        for w in range(16 // WAVE):
            copies = []
            lanes = []
            for j in range(WAVE):
                k = w * WAVE + j
                ub, pb_, nb_ = uv[k], pv[k], nv[k]
                us = pl.multiple_of((ub >> 7) << 7, 128)
                ps = pl.multiple_of((pb_ >> 7) << 7, 128)
                ns = pl.multiple_of((nb_ >> 7) << 7, 128)
                copies.append(pltpu.async_copy(
                    utabT_hbm.at[:, pl.ds(us, 128)], ubuf.at[j], sem))
                copies.append(pltpu.async_copy(
                    itabT_hbm.at[:, pl.ds(ps, 128)], pbuf.at[j], sem))
                copies.append(pltpu.async_copy(
                    itabT_hbm.at[:, pl.ds(ns, 128)], nbuf.at[j], sem))
                lanes.append((ub & 127, pb_ & 127, nb_ & 127))
            for c in copies:
                c.wait()
            for j in range(WAVE):
                k = w * WAVE + j
                ul, pl_, nl = lanes[j]
                j16 = jnp.full((16,), j, jnp.int32)
                ul16 = jnp.full((16,), ul, jnp.int32)
                pl16 = jnp.full((16,), pl_, jnp.int32)
                nl16 = jnp.full((16,), nl, jnp.int32)
                acc = jnp.zeros((16,), jnp.float32)
                for dg in range(D // 16):
                    d16 = dg * 16 + iota16
                    uu = plsc.load_gather(ubuf, [j16, d16, ul16])
                    pp = plsc.load_gather(pbuf, [j16, d16, pl16])
                    nn = plsc.load_gather(nbuf, [j16, d16, nl16])
                    acc = acc + uu * (pp - nn)
                s = jnp.sum(acc)
                diff16 = jnp.where(iota16 == k, s, diff16)
        diffv[pl.ds(g * 16, 16)] = diff16
        return carry

    lax.fori_loop(0, BPW // 16, group_body, 0)
    pltpu.sync_copy(diffv, diff_hbm.at[pl.ds(base, BPW)])


_dot_call = pl.kernel(
    _dot_body,
    out_type=jax.ShapeDtypeStruct((B,), jnp.float32),
    mesh=_MESH,
    scratch_types=[
        pltpu.VMEM((BPW,), jnp.int32),
        pltpu.VMEM((BPW,), jnp.int32),
        pltpu.VMEM((BPW,), jnp.int32),
        pltpu.VMEM((WAVE, D, 128), jnp.float32),
        pltpu.VMEM((WAVE, D, 128), jnp.float32),
        pltpu.VMEM((WAVE, D, 128), jnp.float32),
        pltpu.VMEM((BPW,), jnp.float32),
        pltpu.SemaphoreType.DMA,
    ],
    compiler_params=pltpu.CompilerParams(needs_layout_passes=False),
)


def _bias_body(pid_hbm, nid_hbm, ibias_hbm, bdiff_hbm,
               pidx, nidx, pb, nb, bdiffv, sem):
    base = _worker_base()
    pltpu.sync_copy(pid_hbm.at[pl.ds(base, BPW)], pidx)
    pltpu.sync_copy(nid_hbm.at[pl.ds(base, BPW)], nidx)
    copies = []
    for j in range(BPW // 128):
        sl = pl.ds(j * 128, 128)
        copies.append(pltpu.async_copy(ibias_hbm.at[pidx.at[sl]], pb.at[sl], sem))
        copies.append(pltpu.async_copy(ibias_hbm.at[nidx.at[sl]], nb.at[sl], sem))
    for c in copies:
        c.wait()

    def group_body(i, carry):
        sl = pl.ds(i * 16, 16)
        bdiffv[sl] = pb[sl] - nb[sl]
        return carry

    lax.fori_loop(0, BPW // 16, group_body, 0)
    pltpu.sync_copy(bdiffv, bdiff_hbm.at[pl.ds(base, BPW)])


_bias_call = pl.kernel(
    _bias_body,
    out_type=jax.ShapeDtypeStruct((B,), jnp.float32),
    mesh=_MESH,
    scratch_types=[
        pltpu.VMEM((BPW,), jnp.int32),
        pltpu.VMEM((BPW,), jnp.int32),
        pltpu.VMEM((BPW,), jnp.float32),
        pltpu.VMEM((BPW,), jnp.float32),
        pltpu.VMEM((BPW,), jnp.float32),
        pltpu.SemaphoreType.DMA,
    ],
    compiler_params=pltpu.CompilerParams(needs_layout_passes=False,
                                         use_tc_tiling_on_sc=False),
)


def _loss_body(diff_ref, bd_ref, out_ref):
    x = diff_ref[...] + bd_ref[...]
    sp = jnp.maximum(-x, 0.0) + jnp.log1p(jnp.exp(-jnp.abs(x)))
    out_ref[...] = jnp.sum(sp).reshape(1, 1)


_loss_call = pl.pallas_call(
    _loss_body,
    out_shape=jax.ShapeDtypeStruct((1, 1), jnp.float32),
)


def kernel(user_id, p_item_id, n_item_id, user_table, item_table, item_bias):
    uid = user_id.astype(jnp.int32)
    pid = p_item_id.astype(jnp.int32)
    nid = n_item_id.astype(jnp.int32)
    diff = _dot_call(uid, pid, nid, user_table.T, item_table.T)
    bdiff = _bias_call(pid, nid, item_bias.reshape(-1))
    loss = _loss_call(diff.reshape(B // 128, 128),
                      bdiff.reshape(B // 128, 128))
    return loss[0, 0]
